# baseline (speedup 1.0000x reference)
import functools

import jax
import jax.numpy as jnp
from jax import lax
from jax.experimental import pallas as pl
from jax.experimental.pallas import tpu as pltpu
from jax.experimental.pallas import tpu_sc as plsc

V = 10000
E = 320000
C = 256
H = 56
W = 56
HID = 128

VP = 10240
NC, NS = 2, 16
NW = NC * NS
VPW = VP // NW
ROWS_PER_TILE = VP // NS

GH, GW = H + 2, W + 2
GP = GH * GW
HWP = 3200

CH = 128
IG = 8
ARCS = 2 * E
APT = -(-ARCS // (NW * CH * IG)) * IG
ARCP = NW * CH * APT

BLK = 512


def _table_body(ft_ref, wb_ref, o_ref):
    o_ref[...] = jnp.dot(ft_ref[...], wb_ref[...],
                         preferred_element_type=jnp.float32)


def _make_table(featT, Wb):
    return pl.pallas_call(
        _table_body,
        grid=(HWP // 128,),
        in_specs=[
            pl.BlockSpec((128, C), lambda i: (i, 0)),
            pl.BlockSpec((C, HID), lambda i: (0, 0)),
        ],
        out_specs=pl.BlockSpec((128, HID), lambda i: (i, 0)),
        out_shape=jax.ShapeDtypeStruct((HWP, HID), jnp.float32),
    )(featT, Wb)


def _coords_body(xs_ref, ys_ref, i0_ref, i1_ref, wy0_ref, wy1_ref,
                 wxa_ref, wxb_ref):
    x = (xs_ref[...] + 1.0) * ((W - 1) / 2.0)
    y = (ys_ref[...] + 1.0) * ((H - 1) / 2.0)
    x0 = jnp.floor(x)
    y0 = jnp.floor(y)
    wx1 = x - x0
    wy1 = y - y0
    x0i = x0.astype(jnp.int32)
    y0i = y0.astype(jnp.int32)
    vx0 = (x0i >= 0) & (x0i <= W - 1)
    vx1 = (x0i >= -1) & (x0i <= W - 2)
    vy0 = (y0i >= 0) & (y0i <= H - 1)
    vy1 = (y0i >= -1) & (y0i <= H - 2)
    pc = jnp.clip(x0i + 1, 0, GW - 2)
    py0 = jnp.clip(y0i + 1, 0, GH - 1)
    py1 = jnp.clip(y0i + 2, 0, GH - 1)
    i0_ref[...] = py0 * GW + pc
    i1_ref[...] = py1 * GW + pc
    wy0_ref[...] = jnp.where(vy0, 1.0 - wy1, 0.0)
    wy1_ref[...] = jnp.where(vy1, wy1, 0.0)
    wxa_ref[...] = jnp.where(vx0, 1.0 - wx1, 0.0)
    wxb_ref[...] = jnp.where(vx1, wx1, 0.0)


def _make_coords(xs, ys):
    n = VP // 128
    f32 = jnp.float32
    outs = pl.pallas_call(
        _coords_body,
        grid=(1,),
        in_specs=[pl.BlockSpec((n, 128), lambda i: (0, 0))] * 2,
        out_specs=[pl.BlockSpec((n, 128), lambda i: (0, 0))] * 6,
        out_shape=[
            jax.ShapeDtypeStruct((n, 128), jnp.int32),
            jax.ShapeDtypeStruct((n, 128), jnp.int32),
            jax.ShapeDtypeStruct((n, 128), f32),
            jax.ShapeDtypeStruct((n, 128), f32),
            jax.ShapeDtypeStruct((n, 128), f32),
            jax.ShapeDtypeStruct((n, 128), f32),
        ],
    )(xs.reshape(n, 128), ys.reshape(n, 128))
    return [o.reshape(VP) for o in outs]


def _make_vert_gather(tp, idx0, idx1):
    mesh = plsc.VectorSubcoreMesh(core_axis_name="c", subcore_axis_name="s")

    @functools.partial(
        pl.kernel,
        mesh=mesh,
        out_type=(
            jax.ShapeDtypeStruct((VP, 2 * HID), jnp.float32),
            jax.ShapeDtypeStruct((VP, 2 * HID), jnp.float32),
        ),
        scratch_types=[
            pltpu.VMEM((CH,), jnp.int32),
            pltpu.VMEM((CH, 2 * HID), jnp.float32),
            pltpu.SemaphoreType.DMA,
        ],
    )
    def k(tp_hbm, i0_hbm, i1_hbm, va0_hbm, va1_hbm, idxv, rows, sem):
        wid = lax.axis_index("s") * NC + lax.axis_index("c")
        base = wid * VPW
        for ih, vh in ((i0_hbm, va0_hbm), (i1_hbm, va1_hbm)):
            for off, n in ((0, 128), (128, 128), (256, 64)):
                pltpu.sync_copy(ih.at[pl.ds(base + off, n)],
                                idxv.at[pl.ds(0, n)])
                pltpu.async_copy(tp_hbm.at[idxv.at[pl.ds(0, n)]],
                                 rows.at[pl.ds(0, n)], sem).wait()
                pltpu.sync_copy(rows.at[pl.ds(0, n)],
                                vh.at[pl.ds(base + off, n)])

    return k(tp, idx0, idx1)


def _make_edge_agg(nbr, arcs, zrows):
    mesh = plsc.VectorSubcoreMesh(core_axis_name="c", subcore_axis_name="s")

    @functools.partial(
        pl.kernel,
        mesh=mesh,
        out_type=jax.ShapeDtypeStruct((NC, VP, HID), jnp.float32),
        scratch_types=[
            pltpu.VMEM((IG, 2, CH), jnp.int32),
            pltpu.VMEM((CH, HID), jnp.float32),
            pltpu.VMEM_SHARED((VP, HID), jnp.float32),
            pltpu.SemaphoreType.DMA,
        ],
    )
    def k(nbr_hbm, arcs_hbm, z_hbm, agg_hbm, idxg, rows, acc, sem):
        cid = lax.axis_index("c")
        sid = lax.axis_index("s")
        wid = sid * NC + cid
        pltpu.sync_copy(z_hbm, acc.at[pl.ds(sid * ROWS_PER_TILE,
                                            ROWS_PER_TILE)])
        plsc.subcore_barrier()

        def body(g, carry):
            pltpu.sync_copy(arcs_hbm.at[wid, pl.ds(g * IG, IG)], idxg)
            for b in range(IG):
                pltpu.async_copy(nbr_hbm.at[idxg.at[b, 0]], rows, sem).wait()
                pltpu.sync_copy(rows, acc.at[idxg.at[b, 1]], add=True)
            return carry

        lax.fori_loop(0, APT // IG, body, 0)
        plsc.subcore_barrier()
        pltpu.sync_copy(
            acc.at[pl.ds(sid * ROWS_PER_TILE, ROWS_PER_TILE)],
            agg_hbm.at[cid, pl.ds(sid * ROWS_PER_TILE, ROWS_PER_TILE)])

    return k(nbr, arcs, zrows)


def _layer0_body(va0_ref, va1_ref, wy0_ref, wy1_ref, wxa_ref, wxb_ref,
                 bb_ref, vp_ref, wh_ref, wv_ref, bc_ref, out_ref, nbr_ref):
    t = wy0_ref[...] * va0_ref[...] + wy1_ref[...] * va1_ref[...]
    h = wxa_ref[...] * t[:, :HID] + wxb_ref[...] * t[:, HID:]
    x = jnp.maximum(h + bb_ref[...], 0.0)
    y = (jnp.dot(x, wh_ref[...], preferred_element_type=jnp.float32)
         + jnp.dot(vp_ref[...], wv_ref[...],
                   preferred_element_type=jnp.float32)
         + bc_ref[...])
    out_ref[...] = y[:, :HID]
    nbr_ref[...] = y[:, HID:]


def _layer_body(prev_ref, agg_ref, vp_ref, wh_ref, wv_ref, bc_ref,
                out_ref, nbr_ref):
    x = jnp.maximum(prev_ref[...] + agg_ref[0] + agg_ref[1], 0.0)
    y = (jnp.dot(x, wh_ref[...], preferred_element_type=jnp.float32)
         + jnp.dot(vp_ref[...], wv_ref[...],
                   preferred_element_type=jnp.float32)
         + bc_ref[...])
    out_ref[...] = y[:, :HID]
    nbr_ref[...] = y[:, HID:]


def _final_body(prev_ref, agg_ref, vp_ref, v128_ref, wh_ref, wv_ref,
                bc_ref, nopos_ref, newv_ref):
    x = jnp.maximum(prev_ref[...] + agg_ref[0] + agg_ref[1], 0.0)
    off = (jnp.dot(x, wh_ref[...], preferred_element_type=jnp.float32)
           + jnp.dot(vp_ref[...], wv_ref[...],
                     preferred_element_type=jnp.float32)
           + bc_ref[...])
    off = jnp.where(jnp.isnan(off), 0.0, off)
    col = lax.broadcasted_iota(jnp.int32, off.shape, 1)
    nopos_ref[...] = x
    newv_ref[...] = v128_ref[...] + jnp.where(col < 3, off, 0.0)


def _run_layer0(va0, va1, wy0, wy1, wxa, wxb, bb, vpad8, Wh, Wv, bc):
    vec = lambda: pl.BlockSpec((BLK, 1), lambda i: (i, 0))
    return pl.pallas_call(
        _layer0_body,
        grid=(VP // BLK,),
        in_specs=[
            pl.BlockSpec((BLK, 2 * HID), lambda i: (i, 0)),
            pl.BlockSpec((BLK, 2 * HID), lambda i: (i, 0)),
            vec(), vec(), vec(), vec(),
            pl.BlockSpec((1, HID), lambda i: (0, 0)),
            pl.BlockSpec((BLK, 8), lambda i: (i, 0)),
            pl.BlockSpec((HID, 2 * HID), lambda i: (0, 0)),
            pl.BlockSpec((8, 2 * HID), lambda i: (0, 0)),
            pl.BlockSpec((1, 2 * HID), lambda i: (0, 0)),
        ],
        out_specs=[pl.BlockSpec((BLK, HID), lambda i: (i, 0))] * 2,
        out_shape=[jax.ShapeDtypeStruct((VP, HID), jnp.float32)] * 2,
    )(va0, va1, wy0, wy1, wxa, wxb, bb, vpad8, Wh, Wv, bc)


def _run_layer(prev, agg, vpad8, Wh, Wv, bc):
    return pl.pallas_call(
        _layer_body,
        grid=(VP // BLK,),
        in_specs=[
            pl.BlockSpec((BLK, HID), lambda i: (i, 0)),
            pl.BlockSpec((NC, BLK, HID), lambda i: (0, i, 0)),
            pl.BlockSpec((BLK, 8), lambda i: (i, 0)),
            pl.BlockSpec((HID, 2 * HID), lambda i: (0, 0)),
            pl.BlockSpec((8, 2 * HID), lambda i: (0, 0)),
            pl.BlockSpec((1, 2 * HID), lambda i: (0, 0)),
        ],
        out_specs=[pl.BlockSpec((BLK, HID), lambda i: (i, 0))] * 2,
        out_shape=[jax.ShapeDtypeStruct((VP, HID), jnp.float32)] * 2,
    )(prev, agg, vpad8, Wh, Wv, bc)


def _run_final(prev, agg, vpad8, vpad128, Wh, Wv, bc):
    return pl.pallas_call(
        _final_body,
        grid=(VP // BLK,),
        in_specs=[
            pl.BlockSpec((BLK, HID), lambda i: (i, 0)),
            pl.BlockSpec((NC, BLK, HID), lambda i: (0, i, 0)),
            pl.BlockSpec((BLK, 8), lambda i: (i, 0)),
            pl.BlockSpec((BLK, HID), lambda i: (i, 0)),
            pl.BlockSpec((HID, HID), lambda i: (0, 0)),
            pl.BlockSpec((8, HID), lambda i: (0, 0)),
            pl.BlockSpec((1, HID), lambda i: (0, 0)),
        ],
        out_specs=[pl.BlockSpec((BLK, HID), lambda i: (i, 0))] * 2,
        out_shape=[jax.ShapeDtypeStruct((VP, HID), jnp.float32)] * 2,
    )(prev, agg, vpad8, vpad128, Wh, Wv, bc)


def kernel(img_feats, verts, edges, Wb, bb, gparams, Woff, boff):
    f32 = jnp.float32

    featT = img_feats[0].reshape(C, H * W).T
    featT = jnp.pad(featT, ((0, HWP - H * W), (0, 0)))
    table = _make_table(featT, Wb)

    grid = table[:H * W].reshape(H, W, HID)
    gpad = jnp.pad(grid, ((1, 1), (1, 1), (0, 0))).reshape(GP, HID)
    gshift = jnp.concatenate([gpad[1:], jnp.zeros((1, HID), f32)], axis=0)
    tp = jnp.concatenate([gpad, gshift], axis=1)

    vx = jnp.pad(verts[:, 0], (0, VP - V))
    vy = jnp.pad(verts[:, 1], (0, VP - V))
    idx0, idx1, wy0, wy1, wxa, wxb = _make_coords(vx, vy)

    va0, va1 = _make_vert_gather(tp, idx0, idx1)

    vpad8 = jnp.pad(verts, ((0, VP - V), (0, 5)))
    vpad128 = jnp.pad(verts, ((0, VP - V), (0, HID - 3)))

    asrc = jnp.concatenate([edges[:, 1], edges[:, 0]])
    adst = jnp.concatenate([edges[:, 0], edges[:, 1]])
    asrc = jnp.pad(asrc, (0, ARCP - ARCS),
                   constant_values=VP - 1).reshape(NW, APT, 1, CH)
    adst = jnp.pad(adst, (0, ARCP - ARCS),
                   constant_values=VP - 1).reshape(NW, APT, 1, CH)
    arcs = jnp.concatenate([asrc, adst], axis=2)
    zrows = jnp.zeros((ROWS_PER_TILE, HID), f32)

    col2 = lambda v: v.reshape(VP, 1)

    w0, b0, w1, b1 = gparams[0]
    Wh = jnp.concatenate([w0[:HID], w1[:HID]], axis=1)
    Wv = jnp.pad(jnp.concatenate([w0[HID:], w1[HID:]], axis=1),
                 ((0, 5), (0, 0)))
    bc = jnp.concatenate([b0, b1]).reshape(1, 2 * HID)
    out, nbr = _run_layer0(va0, va1, col2(wy0), col2(wy1), col2(wxa),
                           col2(wxb), bb.reshape(1, HID), vpad8, Wh, Wv, bc)

    for li in range(1, len(gparams)):
        agg = _make_edge_agg(nbr, arcs, zrows)
        w0, b0, w1, b1 = gparams[li]
        Wh = jnp.concatenate([w0[:HID], w1[:HID]], axis=1)
        Wv = jnp.pad(jnp.concatenate([w0[HID:], w1[HID:]], axis=1),
                     ((0, 5), (0, 0)))
        bc = jnp.concatenate([b0, b1]).reshape(1, 2 * HID)
        out, nbr = _run_layer(out, agg, vpad8, Wh, Wv, bc)

    agg = _make_edge_agg(nbr, arcs, zrows)
    Whf = jnp.pad(Woff[:HID], ((0, 0), (0, HID - 3)))
    Wvf = jnp.pad(Woff[HID:], ((0, 5), (0, HID - 3)))
    bcf = jnp.pad(boff, (0, HID - 3)).reshape(1, HID)
    nopos, newv = _run_final(out, agg, vpad8, vpad128, Whf, Wvf, bcf)

    return newv[:V, :3], nopos[:V]


# speedup vs baseline: 1.6266x; 1.6266x over previous
import functools

import jax
import jax.numpy as jnp
from jax import lax
from jax.experimental import pallas as pl
from jax.experimental.pallas import tpu as pltpu
from jax.experimental.pallas import tpu_sc as plsc

V = 10000
E = 320000
C = 256
H = 56
W = 56
HID = 128

VP = 10240
NC, NS = 2, 16
NW = NC * NS
VPW = VP // NW
ROWS_PER_TILE = VP // NS

GH, GW = H + 2, W + 2
GP = GH * GW
HWP = 3200

CH = 128
ARCS = 2 * E
NCHUNK = -(-ARCS // CH)
APT0 = 139
APT1 = 175
APTMAX = max(APT0, APT1)
NCHUNKP = NS * (APT0 + APT1)
ARCP = NCHUNKP * CH

BLK = 512


def _table_body(ft_ref, wb_ref, o_ref):
    o_ref[...] = jnp.dot(ft_ref[...], wb_ref[...],
                         preferred_element_type=jnp.float32)


def _make_table(featT, Wb):
    return pl.pallas_call(
        _table_body,
        grid=(HWP // 128,),
        in_specs=[
            pl.BlockSpec((128, C), lambda i: (i, 0)),
            pl.BlockSpec((C, HID), lambda i: (0, 0)),
        ],
        out_specs=pl.BlockSpec((128, HID), lambda i: (i, 0)),
        out_shape=jax.ShapeDtypeStruct((HWP, HID), jnp.float32),
    )(featT, Wb)


def _coords_body(xs_ref, ys_ref, i0_ref, i1_ref, wy0_ref, wy1_ref,
                 wxa_ref, wxb_ref):
    x = (xs_ref[...] + 1.0) * ((W - 1) / 2.0)
    y = (ys_ref[...] + 1.0) * ((H - 1) / 2.0)
    x0 = jnp.floor(x)
    y0 = jnp.floor(y)
    wx1 = x - x0
    wy1 = y - y0
    x0i = x0.astype(jnp.int32)
    y0i = y0.astype(jnp.int32)
    vx0 = (x0i >= 0) & (x0i <= W - 1)
    vx1 = (x0i >= -1) & (x0i <= W - 2)
    vy0 = (y0i >= 0) & (y0i <= H - 1)
    vy1 = (y0i >= -1) & (y0i <= H - 2)
    pc = jnp.clip(x0i + 1, 0, GW - 2)
    py0 = jnp.clip(y0i + 1, 0, GH - 1)
    py1 = jnp.clip(y0i + 2, 0, GH - 1)
    i0_ref[...] = py0 * GW + pc
    i1_ref[...] = py1 * GW + pc
    wy0_ref[...] = jnp.where(vy0, 1.0 - wy1, 0.0)
    wy1_ref[...] = jnp.where(vy1, wy1, 0.0)
    wxa_ref[...] = jnp.where(vx0, 1.0 - wx1, 0.0)
    wxb_ref[...] = jnp.where(vx1, wx1, 0.0)


def _make_coords(xs, ys):
    n = VP // 128
    f32 = jnp.float32
    outs = pl.pallas_call(
        _coords_body,
        grid=(1,),
        in_specs=[pl.BlockSpec((n, 128), lambda i: (0, 0))] * 2,
        out_specs=[pl.BlockSpec((n, 128), lambda i: (0, 0))] * 6,
        out_shape=[
            jax.ShapeDtypeStruct((n, 128), jnp.int32),
            jax.ShapeDtypeStruct((n, 128), jnp.int32),
            jax.ShapeDtypeStruct((n, 128), f32),
            jax.ShapeDtypeStruct((n, 128), f32),
            jax.ShapeDtypeStruct((n, 128), f32),
            jax.ShapeDtypeStruct((n, 128), f32),
        ],
    )(xs.reshape(n, 128), ys.reshape(n, 128))
    return [o.reshape(VP) for o in outs]


VPWP = 384


def _make_vert_gather(tp, i01):
    mesh = plsc.VectorSubcoreMesh(core_axis_name="c", subcore_axis_name="s")

    @functools.partial(
        pl.kernel,
        mesh=mesh,
        out_type=(
            jax.ShapeDtypeStruct((VP, 2 * HID), jnp.float32),
            jax.ShapeDtypeStruct((VP, 2 * HID), jnp.float32),
        ),
        scratch_types=[
            pltpu.VMEM((2, VPWP), jnp.int32),
            pltpu.VMEM((VPW, 2 * HID), jnp.float32),
            pltpu.SemaphoreType.DMA,
        ],
    )
    def k(tp_hbm, i01_hbm, va0_hbm, va1_hbm, ibuf, rows, sem):
        wid = lax.axis_index("s") * NC + lax.axis_index("c")
        base = wid * VPW
        for j in range(2):
            pltpu.sync_copy(i01_hbm.at[pl.ds((j * NW + wid) * VPWP, VPWP)],
                            ibuf.at[j])
        for j, vh in ((0, va0_hbm), (1, va1_hbm)):
            for off, n in ((0, 128), (128, 128), (256, 64)):
                pltpu.async_copy(tp_hbm.at[ibuf.at[j, pl.ds(off, n)]],
                                 rows.at[pl.ds(off, n)], sem).wait()
            pltpu.sync_copy(rows, vh.at[pl.ds(base, VPW)])

    return k(tp, i01)


def _make_edge_agg(nbr, arcs, zrows):
    mesh = plsc.VectorSubcoreMesh(core_axis_name="c", subcore_axis_name="s")

    @functools.partial(
        pl.kernel,
        mesh=mesh,
        out_type=jax.ShapeDtypeStruct((NC, VP, HID), jnp.float32),
        scratch_types=[
            pltpu.VMEM((2, CH), jnp.int32),
            pltpu.VMEM((CH, HID), jnp.float32),
            pltpu.VMEM_SHARED((VP, HID), jnp.float32),
            pltpu.SemaphoreType.DMA,
        ],
    )
    def k(nbr_hbm, arcs_hbm, z_hbm, agg_hbm, idxv, rows, acc, sem):
        cid = lax.axis_index("c")
        sid = lax.axis_index("s")
        wid = sid * NC + cid
        pltpu.sync_copy(z_hbm, acc.at[pl.ds(sid * ROWS_PER_TILE,
                                            ROWS_PER_TILE)])
        plsc.subcore_barrier()

        def body(i, carry):
            pltpu.sync_copy(arcs_hbm.at[wid, i], idxv)
            pltpu.async_copy(nbr_hbm.at[idxv.at[0]], rows, sem).wait()
            pltpu.sync_copy(rows, acc.at[idxv.at[1]], add=True)
            return carry

        lax.fori_loop(0, jnp.where(cid == 0, APT0, APT1), body, 0)
        plsc.subcore_barrier()
        pltpu.sync_copy(
            acc.at[pl.ds(sid * ROWS_PER_TILE, ROWS_PER_TILE)],
            agg_hbm.at[cid, pl.ds(sid * ROWS_PER_TILE, ROWS_PER_TILE)])

    return k(nbr, arcs, zrows)


def _layer0_body(va0_ref, va1_ref, wy0_ref, wy1_ref, wxa_ref, wxb_ref,
                 bb_ref, vp_ref, wh_ref, wv_ref, bc_ref, out_ref, nbr_ref):
    t = wy0_ref[...] * va0_ref[...] + wy1_ref[...] * va1_ref[...]
    h = wxa_ref[...] * t[:, :HID] + wxb_ref[...] * t[:, HID:]
    x = jnp.maximum(h + bb_ref[...], 0.0)
    y = (jnp.dot(x, wh_ref[...], preferred_element_type=jnp.float32)
         + jnp.dot(vp_ref[...], wv_ref[...],
                   preferred_element_type=jnp.float32)
         + bc_ref[...])
    out_ref[...] = y[:, :HID]
    nbr_ref[...] = y[:, HID:]


def _layer_body(prev_ref, agg_ref, vp_ref, wh_ref, wv_ref, bc_ref,
                out_ref, nbr_ref):
    x = jnp.maximum(prev_ref[...] + agg_ref[0] + agg_ref[1], 0.0)
    y = (jnp.dot(x, wh_ref[...], preferred_element_type=jnp.float32)
         + jnp.dot(vp_ref[...], wv_ref[...],
                   preferred_element_type=jnp.float32)
         + bc_ref[...])
    out_ref[...] = y[:, :HID]
    nbr_ref[...] = y[:, HID:]


def _final_body(prev_ref, agg_ref, vp_ref, v128_ref, wh_ref, wv_ref,
                bc_ref, nopos_ref, newv_ref):
    x = jnp.maximum(prev_ref[...] + agg_ref[0] + agg_ref[1], 0.0)
    off = (jnp.dot(x, wh_ref[...], preferred_element_type=jnp.float32)
           + jnp.dot(vp_ref[...], wv_ref[...],
                     preferred_element_type=jnp.float32)
           + bc_ref[...])
    off = jnp.where(jnp.isnan(off), 0.0, off)
    col = lax.broadcasted_iota(jnp.int32, off.shape, 1)
    nopos_ref[...] = x
    newv_ref[...] = v128_ref[...] + jnp.where(col < 3, off, 0.0)


def _run_layer0(va0, va1, wy0, wy1, wxa, wxb, bb, vpad8, Wh, Wv, bc):
    vec = lambda: pl.BlockSpec((BLK, 1), lambda i: (i, 0))
    return pl.pallas_call(
        _layer0_body,
        grid=(VP // BLK,),
        in_specs=[
            pl.BlockSpec((BLK, 2 * HID), lambda i: (i, 0)),
            pl.BlockSpec((BLK, 2 * HID), lambda i: (i, 0)),
            vec(), vec(), vec(), vec(),
            pl.BlockSpec((1, HID), lambda i: (0, 0)),
            pl.BlockSpec((BLK, 8), lambda i: (i, 0)),
            pl.BlockSpec((HID, 2 * HID), lambda i: (0, 0)),
            pl.BlockSpec((8, 2 * HID), lambda i: (0, 0)),
            pl.BlockSpec((1, 2 * HID), lambda i: (0, 0)),
        ],
        out_specs=[pl.BlockSpec((BLK, HID), lambda i: (i, 0))] * 2,
        out_shape=[jax.ShapeDtypeStruct((VP, HID), jnp.float32)] * 2,
    )(va0, va1, wy0, wy1, wxa, wxb, bb, vpad8, Wh, Wv, bc)


def _run_layer(prev, agg, vpad8, Wh, Wv, bc):
    return pl.pallas_call(
        _layer_body,
        grid=(VP // BLK,),
        in_specs=[
            pl.BlockSpec((BLK, HID), lambda i: (i, 0)),
            pl.BlockSpec((NC, BLK, HID), lambda i: (0, i, 0)),
            pl.BlockSpec((BLK, 8), lambda i: (i, 0)),
            pl.BlockSpec((HID, 2 * HID), lambda i: (0, 0)),
            pl.BlockSpec((8, 2 * HID), lambda i: (0, 0)),
            pl.BlockSpec((1, 2 * HID), lambda i: (0, 0)),
        ],
        out_specs=[pl.BlockSpec((BLK, HID), lambda i: (i, 0))] * 2,
        out_shape=[jax.ShapeDtypeStruct((VP, HID), jnp.float32)] * 2,
    )(prev, agg, vpad8, Wh, Wv, bc)


def _run_final(prev, agg, vpad8, vpad128, Wh, Wv, bc):
    return pl.pallas_call(
        _final_body,
        grid=(VP // BLK,),
        in_specs=[
            pl.BlockSpec((BLK, HID), lambda i: (i, 0)),
            pl.BlockSpec((NC, BLK, HID), lambda i: (0, i, 0)),
            pl.BlockSpec((BLK, 8), lambda i: (i, 0)),
            pl.BlockSpec((BLK, HID), lambda i: (i, 0)),
            pl.BlockSpec((HID, HID), lambda i: (0, 0)),
            pl.BlockSpec((8, HID), lambda i: (0, 0)),
            pl.BlockSpec((1, HID), lambda i: (0, 0)),
        ],
        out_specs=[pl.BlockSpec((BLK, HID), lambda i: (i, 0))] * 2,
        out_shape=[jax.ShapeDtypeStruct((VP, HID), jnp.float32)] * 2,
    )(prev, agg, vpad8, vpad128, Wh, Wv, bc)


def kernel(img_feats, verts, edges, Wb, bb, gparams, Woff, boff):
    f32 = jnp.float32

    featT = img_feats[0].reshape(C, H * W).T
    featT = jnp.pad(featT, ((0, HWP - H * W), (0, 0)))
    table = _make_table(featT, Wb)

    grid = table[:H * W].reshape(H, W, HID)
    gpad = jnp.pad(grid, ((1, 1), (1, 1), (0, 0))).reshape(GP, HID)
    gshift = jnp.concatenate([gpad[1:], jnp.zeros((1, HID), f32)], axis=0)
    tp = jnp.concatenate([gpad, gshift], axis=1)

    vx = jnp.pad(verts[:, 0], (0, VP - V))
    vy = jnp.pad(verts[:, 1], (0, VP - V))
    idx0, idx1, wy0, wy1, wxa, wxb = _make_coords(vx, vy)

    pad_idx = lambda a: jnp.pad(a.reshape(NW, VPW),
                                ((0, 0), (0, VPWP - VPW))).reshape(-1)
    va0, va1 = _make_vert_gather(
        tp, jnp.concatenate([pad_idx(idx0), pad_idx(idx1)]))

    vpad8 = jnp.pad(verts, ((0, VP - V), (0, 5)))
    vpad128 = jnp.pad(verts, ((0, VP - V), (0, HID - 3)))

    asrc = jnp.concatenate([edges[:, 1], edges[:, 0]])
    adst = jnp.concatenate([edges[:, 0], edges[:, 1]])
    asrc = jnp.pad(asrc, (0, ARCP - ARCS),
                   constant_values=VP - 1).reshape(NCHUNKP, 1, CH)
    adst = jnp.pad(adst, (0, ARCP - ARCS),
                   constant_values=VP - 1).reshape(NCHUNKP, 1, CH)
    chunks = jnp.concatenate([asrc, adst], axis=1)
    junk = jnp.full((1, 2, CH), VP - 1, jnp.int32)
    pieces, off = [], 0
    for w in range(NW):
        cnt = APT0 if w % NC == 0 else APT1
        piece = chunks[off:off + cnt]
        if cnt < APTMAX:
            piece = jnp.concatenate(
                [piece, jnp.broadcast_to(junk, (APTMAX - cnt, 2, CH))])
        pieces.append(piece)
        off += cnt
    arcs = jnp.stack(pieces)
    zrows = jnp.zeros((ROWS_PER_TILE, HID), f32)

    col2 = lambda v: v.reshape(VP, 1)

    w0, b0, w1, b1 = gparams[0]
    Wh = jnp.concatenate([w0[:HID], w1[:HID]], axis=1)
    Wv = jnp.pad(jnp.concatenate([w0[HID:], w1[HID:]], axis=1),
                 ((0, 5), (0, 0)))
    bc = jnp.concatenate([b0, b1]).reshape(1, 2 * HID)
    out, nbr = _run_layer0(va0, va1, col2(wy0), col2(wy1), col2(wxa),
                           col2(wxb), bb.reshape(1, HID), vpad8, Wh, Wv, bc)

    for li in range(1, len(gparams)):
        agg = _make_edge_agg(nbr, arcs, zrows)
        w0, b0, w1, b1 = gparams[li]
        Wh = jnp.concatenate([w0[:HID], w1[:HID]], axis=1)
        Wv = jnp.pad(jnp.concatenate([w0[HID:], w1[HID:]], axis=1),
                     ((0, 5), (0, 0)))
        bc = jnp.concatenate([b0, b1]).reshape(1, 2 * HID)
        out, nbr = _run_layer(out, agg, vpad8, Wh, Wv, bc)

    agg = _make_edge_agg(nbr, arcs, zrows)
    Whf = jnp.pad(Woff[:HID], ((0, 0), (0, HID - 3)))
    Wvf = jnp.pad(Woff[HID:], ((0, 5), (0, HID - 3)))
    bcf = jnp.pad(boff, (0, HID - 3)).reshape(1, HID)
    nopos, newv = _run_final(out, agg, vpad8, vpad128, Whf, Wvf, bcf)

    return newv[:V, :3], nopos[:V]


# speedup vs baseline: 1.8755x; 1.1530x over previous
import functools

import jax
import jax.numpy as jnp
from jax import lax
from jax.experimental import pallas as pl
from jax.experimental.pallas import tpu as pltpu
from jax.experimental.pallas import tpu_sc as plsc

V = 10000
E = 320000
C = 256
H = 56
W = 56
HID = 128

VP = 10240
NC, NS = 2, 16
NW = NC * NS
VPW = VP // NW
ROWS_PER_TILE = VP // NS

GH, GW = H + 2, W + 2
GP = GH * GW
HWP = 3200

CH = 128
ARCS = 2 * E
NCHUNK = -(-ARCS // CH)
APT0 = 173
APT1 = 141
APTMAX = max(APT0, APT1)
NCHUNKP = NS * (APT0 + APT1)
ARCP = NCHUNKP * CH

BLK = 512


def _table_body(ft_ref, wb_ref, o_ref):
    o_ref[...] = jnp.dot(ft_ref[...], wb_ref[...],
                         preferred_element_type=jnp.float32)


def _make_table(featT, Wb):
    return pl.pallas_call(
        _table_body,
        grid=(HWP // 128,),
        in_specs=[
            pl.BlockSpec((128, C), lambda i: (i, 0)),
            pl.BlockSpec((C, HID), lambda i: (0, 0)),
        ],
        out_specs=pl.BlockSpec((128, HID), lambda i: (i, 0)),
        out_shape=jax.ShapeDtypeStruct((HWP, HID), jnp.float32),
    )(featT, Wb)


def _coords_body(xs_ref, ys_ref, i0_ref, i1_ref, wy0_ref, wy1_ref,
                 wxa_ref, wxb_ref):
    x = (xs_ref[...] + 1.0) * ((W - 1) / 2.0)
    y = (ys_ref[...] + 1.0) * ((H - 1) / 2.0)
    x0 = jnp.floor(x)
    y0 = jnp.floor(y)
    wx1 = x - x0
    wy1 = y - y0
    x0i = x0.astype(jnp.int32)
    y0i = y0.astype(jnp.int32)
    vx0 = (x0i >= 0) & (x0i <= W - 1)
    vx1 = (x0i >= -1) & (x0i <= W - 2)
    vy0 = (y0i >= 0) & (y0i <= H - 1)
    vy1 = (y0i >= -1) & (y0i <= H - 2)
    pc = jnp.clip(x0i + 1, 0, GW - 2)
    py0 = jnp.clip(y0i + 1, 0, GH - 1)
    py1 = jnp.clip(y0i + 2, 0, GH - 1)
    i0_ref[...] = py0 * GW + pc
    i1_ref[...] = py1 * GW + pc
    wy0_ref[...] = jnp.where(vy0, 1.0 - wy1, 0.0)
    wy1_ref[...] = jnp.where(vy1, wy1, 0.0)
    wxa_ref[...] = jnp.where(vx0, 1.0 - wx1, 0.0)
    wxb_ref[...] = jnp.where(vx1, wx1, 0.0)


def _make_coords(xs, ys):
    n = VP // 128
    f32 = jnp.float32
    outs = pl.pallas_call(
        _coords_body,
        grid=(1,),
        in_specs=[pl.BlockSpec((n, 128), lambda i: (0, 0))] * 2,
        out_specs=[pl.BlockSpec((n, 128), lambda i: (0, 0))] * 6,
        out_shape=[
            jax.ShapeDtypeStruct((n, 128), jnp.int32),
            jax.ShapeDtypeStruct((n, 128), jnp.int32),
            jax.ShapeDtypeStruct((n, 128), f32),
            jax.ShapeDtypeStruct((n, 128), f32),
            jax.ShapeDtypeStruct((n, 128), f32),
            jax.ShapeDtypeStruct((n, 128), f32),
        ],
    )(xs.reshape(n, 128), ys.reshape(n, 128))
    return [o.reshape(VP) for o in outs]


VPWP = 384


def _make_vert_gather(tp, i01):
    mesh = plsc.VectorSubcoreMesh(core_axis_name="c", subcore_axis_name="s")

    @functools.partial(
        pl.kernel,
        mesh=mesh,
        out_type=(
            jax.ShapeDtypeStruct((VP, 2 * HID), jnp.float32),
            jax.ShapeDtypeStruct((VP, 2 * HID), jnp.float32),
        ),
        scratch_types=[
            pltpu.VMEM((2, VPWP), jnp.int32),
            pltpu.VMEM((VPW, 2 * HID), jnp.float32),
            pltpu.SemaphoreType.DMA,
        ],
    )
    def k(tp_hbm, i01_hbm, va0_hbm, va1_hbm, ibuf, rows, sem):
        wid = lax.axis_index("s") * NC + lax.axis_index("c")
        base = wid * VPW
        for j in range(2):
            pltpu.sync_copy(i01_hbm.at[pl.ds((j * NW + wid) * VPWP, VPWP)],
                            ibuf.at[j])
        for j, vh in ((0, va0_hbm), (1, va1_hbm)):
            for off, n in ((0, 128), (128, 128), (256, 64)):
                pltpu.async_copy(tp_hbm.at[ibuf.at[j, pl.ds(off, n)]],
                                 rows.at[pl.ds(off, n)], sem).wait()
            pltpu.sync_copy(rows, vh.at[pl.ds(base, VPW)])

    return k(tp, i01)


def _make_edge_agg(nbr, arcs, zrows):
    mesh = plsc.VectorSubcoreMesh(core_axis_name="c", subcore_axis_name="s")

    @functools.partial(
        pl.kernel,
        mesh=mesh,
        out_type=jax.ShapeDtypeStruct((NC, VP, HID), jnp.float32),
        scratch_types=[
            pltpu.VMEM((2, CH), jnp.int32),
            pltpu.VMEM((CH, HID), jnp.float32),
            pltpu.VMEM_SHARED((VP, HID), jnp.float32),
            pltpu.SemaphoreType.DMA,
        ],
    )
    def k(nbr_hbm, arcs_hbm, z_hbm, agg_hbm, idxv, rows, acc, sem):
        cid = lax.axis_index("c")
        sid = lax.axis_index("s")
        wid = sid * NC + cid
        pltpu.sync_copy(z_hbm, acc.at[pl.ds(sid * ROWS_PER_TILE,
                                            ROWS_PER_TILE)])
        plsc.subcore_barrier()

        def body(i, carry):
            pltpu.sync_copy(arcs_hbm.at[wid, i], idxv)
            pltpu.async_copy(nbr_hbm.at[idxv.at[0]], rows, sem).wait()
            pltpu.sync_copy(rows, acc.at[idxv.at[1]], add=True)
            return carry

        lax.fori_loop(0, jnp.where(cid == 0, APT0, APT1), body, 0)
        plsc.subcore_barrier()
        pltpu.sync_copy(
            acc.at[pl.ds(sid * ROWS_PER_TILE, ROWS_PER_TILE)],
            agg_hbm.at[cid, pl.ds(sid * ROWS_PER_TILE, ROWS_PER_TILE)])

    return k(nbr, arcs, zrows)


def _layer0_body(va0_ref, va1_ref, wy0_ref, wy1_ref, wxa_ref, wxb_ref,
                 bb_ref, vp_ref, wh_ref, wv_ref, bc_ref, out_ref, nbr_ref):
    t = wy0_ref[...] * va0_ref[...] + wy1_ref[...] * va1_ref[...]
    h = wxa_ref[...] * t[:, :HID] + wxb_ref[...] * t[:, HID:]
    x = jnp.maximum(h + bb_ref[...], 0.0)
    y = (jnp.dot(x, wh_ref[...], preferred_element_type=jnp.float32)
         + jnp.dot(vp_ref[...], wv_ref[...],
                   preferred_element_type=jnp.float32)
         + bc_ref[...])
    out_ref[...] = y[:, :HID]
    nbr_ref[...] = y[:, HID:]


def _layer_body(prev_ref, agg_ref, vp_ref, wh_ref, wv_ref, bc_ref,
                out_ref, nbr_ref):
    x = jnp.maximum(prev_ref[...] + agg_ref[0] + agg_ref[1], 0.0)
    y = (jnp.dot(x, wh_ref[...], preferred_element_type=jnp.float32)
         + jnp.dot(vp_ref[...], wv_ref[...],
                   preferred_element_type=jnp.float32)
         + bc_ref[...])
    out_ref[...] = y[:, :HID]
    nbr_ref[...] = y[:, HID:]


def _final_body(prev_ref, agg_ref, vp_ref, v128_ref, wh_ref, wv_ref,
                bc_ref, nopos_ref, newv_ref):
    x = jnp.maximum(prev_ref[...] + agg_ref[0] + agg_ref[1], 0.0)
    off = (jnp.dot(x, wh_ref[...], preferred_element_type=jnp.float32)
           + jnp.dot(vp_ref[...], wv_ref[...],
                     preferred_element_type=jnp.float32)
           + bc_ref[...])
    off = jnp.where(jnp.isnan(off), 0.0, off)
    col = lax.broadcasted_iota(jnp.int32, off.shape, 1)
    nopos_ref[...] = x
    newv_ref[...] = v128_ref[...] + jnp.where(col < 3, off, 0.0)


def _run_layer0(va0, va1, wy0, wy1, wxa, wxb, bb, vpad8, Wh, Wv, bc):
    vec = lambda: pl.BlockSpec((BLK, 1), lambda i: (i, 0))
    return pl.pallas_call(
        _layer0_body,
        grid=(VP // BLK,),
        in_specs=[
            pl.BlockSpec((BLK, 2 * HID), lambda i: (i, 0)),
            pl.BlockSpec((BLK, 2 * HID), lambda i: (i, 0)),
            vec(), vec(), vec(), vec(),
            pl.BlockSpec((1, HID), lambda i: (0, 0)),
            pl.BlockSpec((BLK, 8), lambda i: (i, 0)),
            pl.BlockSpec((HID, 2 * HID), lambda i: (0, 0)),
            pl.BlockSpec((8, 2 * HID), lambda i: (0, 0)),
            pl.BlockSpec((1, 2 * HID), lambda i: (0, 0)),
        ],
        out_specs=[pl.BlockSpec((BLK, HID), lambda i: (i, 0))] * 2,
        out_shape=[jax.ShapeDtypeStruct((VP, HID), jnp.float32)] * 2,
    )(va0, va1, wy0, wy1, wxa, wxb, bb, vpad8, Wh, Wv, bc)


def _run_layer(prev, agg, vpad8, Wh, Wv, bc):
    return pl.pallas_call(
        _layer_body,
        grid=(VP // BLK,),
        in_specs=[
            pl.BlockSpec((BLK, HID), lambda i: (i, 0)),
            pl.BlockSpec((NC, BLK, HID), lambda i: (0, i, 0)),
            pl.BlockSpec((BLK, 8), lambda i: (i, 0)),
            pl.BlockSpec((HID, 2 * HID), lambda i: (0, 0)),
            pl.BlockSpec((8, 2 * HID), lambda i: (0, 0)),
            pl.BlockSpec((1, 2 * HID), lambda i: (0, 0)),
        ],
        out_specs=[pl.BlockSpec((BLK, HID), lambda i: (i, 0))] * 2,
        out_shape=[jax.ShapeDtypeStruct((VP, HID), jnp.float32)] * 2,
    )(prev, agg, vpad8, Wh, Wv, bc)


def _run_final(prev, agg, vpad8, vpad128, Wh, Wv, bc):
    return pl.pallas_call(
        _final_body,
        grid=(VP // BLK,),
        in_specs=[
            pl.BlockSpec((BLK, HID), lambda i: (i, 0)),
            pl.BlockSpec((NC, BLK, HID), lambda i: (0, i, 0)),
            pl.BlockSpec((BLK, 8), lambda i: (i, 0)),
            pl.BlockSpec((BLK, HID), lambda i: (i, 0)),
            pl.BlockSpec((HID, HID), lambda i: (0, 0)),
            pl.BlockSpec((8, HID), lambda i: (0, 0)),
            pl.BlockSpec((1, HID), lambda i: (0, 0)),
        ],
        out_specs=[pl.BlockSpec((BLK, HID), lambda i: (i, 0))] * 2,
        out_shape=[jax.ShapeDtypeStruct((VP, HID), jnp.float32)] * 2,
    )(prev, agg, vpad8, vpad128, Wh, Wv, bc)


def kernel(img_feats, verts, edges, Wb, bb, gparams, Woff, boff):
    f32 = jnp.float32

    featT = img_feats[0].reshape(C, H * W).T
    featT = jnp.pad(featT, ((0, HWP - H * W), (0, 0)))
    table = _make_table(featT, Wb)

    grid = table[:H * W].reshape(H, W, HID)
    gpad = jnp.pad(grid, ((1, 1), (1, 1), (0, 0))).reshape(GP, HID)
    gshift = jnp.concatenate([gpad[1:], jnp.zeros((1, HID), f32)], axis=0)
    tp = jnp.concatenate([gpad, gshift], axis=1)

    vx = jnp.pad(verts[:, 0], (0, VP - V))
    vy = jnp.pad(verts[:, 1], (0, VP - V))
    idx0, idx1, wy0, wy1, wxa, wxb = _make_coords(vx, vy)

    pad_idx = lambda a: jnp.pad(a.reshape(NW, VPW),
                                ((0, 0), (0, VPWP - VPW))).reshape(-1)
    va0, va1 = _make_vert_gather(
        tp, jnp.concatenate([pad_idx(idx0), pad_idx(idx1)]))

    vpad8 = jnp.pad(verts, ((0, VP - V), (0, 5)))
    vpad128 = jnp.pad(verts, ((0, VP - V), (0, HID - 3)))

    asrc = jnp.concatenate([edges[:, 1], edges[:, 0]])
    adst = jnp.concatenate([edges[:, 0], edges[:, 1]])
    asrc = jnp.pad(asrc, (0, ARCP - ARCS),
                   constant_values=VP - 1).reshape(NCHUNKP, 1, CH)
    adst = jnp.pad(adst, (0, ARCP - ARCS),
                   constant_values=VP - 1).reshape(NCHUNKP, 1, CH)
    chunks = jnp.concatenate([asrc, adst], axis=1)
    junk = jnp.full((1, 2, CH), VP - 1, jnp.int32)
    pieces, off = [], 0
    for w in range(NW):
        cnt = APT0 if w % NC == 0 else APT1
        piece = chunks[off:off + cnt]
        if cnt < APTMAX:
            piece = jnp.concatenate(
                [piece, jnp.broadcast_to(junk, (APTMAX - cnt, 2, CH))])
        pieces.append(piece)
        off += cnt
    arcs = jnp.stack(pieces)
    zrows = jnp.zeros((ROWS_PER_TILE, HID), f32)

    col2 = lambda v: v.reshape(VP, 1)

    w0, b0, w1, b1 = gparams[0]
    Wh = jnp.concatenate([w0[:HID], w1[:HID]], axis=1)
    Wv = jnp.pad(jnp.concatenate([w0[HID:], w1[HID:]], axis=1),
                 ((0, 5), (0, 0)))
    bc = jnp.concatenate([b0, b1]).reshape(1, 2 * HID)
    out, nbr = _run_layer0(va0, va1, col2(wy0), col2(wy1), col2(wxa),
                           col2(wxb), bb.reshape(1, HID), vpad8, Wh, Wv, bc)

    for li in range(1, len(gparams)):
        agg = _make_edge_agg(nbr, arcs, zrows)
        w0, b0, w1, b1 = gparams[li]
        Wh = jnp.concatenate([w0[:HID], w1[:HID]], axis=1)
        Wv = jnp.pad(jnp.concatenate([w0[HID:], w1[HID:]], axis=1),
                     ((0, 5), (0, 0)))
        bc = jnp.concatenate([b0, b1]).reshape(1, 2 * HID)
        out, nbr = _run_layer(out, agg, vpad8, Wh, Wv, bc)

    agg = _make_edge_agg(nbr, arcs, zrows)
    Whf = jnp.pad(Woff[:HID], ((0, 0), (0, HID - 3)))
    Wvf = jnp.pad(Woff[HID:], ((0, 5), (0, HID - 3)))
    bcf = jnp.pad(boff, (0, HID - 3)).reshape(1, HID)
    nopos, newv = _run_final(out, agg, vpad8, vpad128, Whf, Wvf, bcf)

    return newv[:V, :3], nopos[:V]
